# baseline (speedup 1.0000x reference)
import functools

import jax
import jax.numpy as jnp
from jax import lax
from jax.experimental import pallas as pl
from jax.experimental.pallas import tpu as pltpu
from jax.experimental.pallas import tpu_sc as plsc

KTAB = 8192
BATCH = 4096
D = 8192
NC, NS, L = 2, 16, 16
NW = NC * NS
BPW = BATCH // NW
TCROWS = 2048
BPWA = (BATCH - TCROWS) // NW
CH = 2
NCHUNK = BPWA // CH
NBUF = 4
UNROLL = 8

_MESH = plsc.VectorSubcoreMesh(core_axis_name="c", subcore_axis_name="s")


def _wid():
    return lax.axis_index("s") * NC + lax.axis_index("c")


def _all_sum(v):
    for sh in (1, 2, 4, 8):
        idx = (lax.iota(jnp.int32, L) + sh) & (L - 1)
        v = v + v.at[idx].get(mode="promise_in_bounds")
    return v


def _gather(probs_hbm, idx_v, c, buf, sem):
    return pltpu.make_async_copy(probs_hbm.at[idx_v.at[c]], buf, sem)


_SCRATCH = [
    pltpu.VMEM((NCHUNK, CH), jnp.int32),
    pltpu.VMEM((CH, D), jnp.float32),
    pltpu.VMEM((CH, D), jnp.float32),
    pltpu.VMEM((CH, D), jnp.float32),
    pltpu.VMEM((CH, D), jnp.float32),
    pltpu.VMEM((L,), jnp.float32),
    pltpu.SemaphoreType.DMA,
    pltpu.SemaphoreType.DMA,
    pltpu.SemaphoreType.DMA,
    pltpu.SemaphoreType.DMA,
]


@functools.partial(
    pl.kernel,
    out_type=jax.ShapeDtypeStruct((4, 128), jnp.float32),
    mesh=_MESH,
    scratch_types=_SCRATCH,
)
def _sum_kernel(probs_hbm, x3_hbm, u_hbm, part_hbm,
                idx_v, b0, b1, b2, b3, vec_v, g0, g1, g2, g3):
    bufs = (b0, b1, b2, b3)
    gsem = (g0, g1, g2, g3)
    wid = _wid()
    pltpu.sync_copy(x3_hbm.at[wid], idx_v)
    pltpu.sync_copy(u_hbm, vec_v)
    s_u = _all_sum(vec_v[...])
    a = 1.0 - s_u
    b = s_u * (1.0 / KTAB)

    for c in range(NBUF - 1):
        _gather(probs_hbm, idx_v, c, bufs[c], gsem[c]).start()

    def step(k, accs):
        for j in range(NBUF):
            c = NBUF * k + j
            _gather(probs_hbm, idx_v, c, bufs[j], gsem[j]).wait()

            def make_row(r, j=j):
                def body(jj, accs):
                    out = []
                    for q in range(UNROLL):
                        v = bufs[j][r, pl.ds((jj * UNROLL + q) * L, L)]
                        z = jnp.minimum(jnp.maximum(a * v + b, 1e-6), 1.0)
                        out.append(accs[q] + z)
                    return tuple(out)
                return body

            for r in range(CH):
                accs = lax.fori_loop(0, D // (L * UNROLL), make_row(r), accs)

            nc = c + NBUF - 1
            nj = (j + NBUF - 1) % NBUF

            @pl.when(nc < NCHUNK)
            def _():
                _gather(probs_hbm, idx_v, nc, bufs[nj], gsem[nj]).start()
        return accs

    zero = jnp.zeros((L,), jnp.float32)
    accs = lax.fori_loop(0, NCHUNK // NBUF, step, (zero,) * UNROLL)
    acc = accs[0]
    for q in range(1, UNROLL):
        acc = acc + accs[q]
    vec_v[...] = acc
    pltpu.sync_copy(vec_v, part_hbm.at[wid // 8, pl.ds((wid % 8) * L, L)])


def _tc_sum_body(x_ref, row_ref, u_ref, acc_ref):
    i = pl.program_id(0)
    s_u = jnp.sum(u_ref[...])
    a = 1.0 - s_u
    b = s_u * (1.0 / KTAB)
    z = jnp.minimum(jnp.maximum(a * row_ref[...] + b, 1e-6), 1.0)
    zz = jnp.sum(z.reshape(8, 8, 128), axis=0)

    @pl.when(i == 0)
    def _():
        acc_ref[...] = jnp.zeros((8, 128), jnp.float32)

    acc_ref[...] += zz


_tc_sum = pl.pallas_call(
    _tc_sum_body,
    grid_spec=pltpu.PrefetchScalarGridSpec(
        num_scalar_prefetch=1,
        grid=(TCROWS,),
        in_specs=[
            pl.BlockSpec((1, 1, D), lambda i, xref: (xref[i], 0, 0)),
            pl.BlockSpec((1, L), lambda i, xref: (0, 0)),
        ],
        out_specs=pl.BlockSpec((8, 128), lambda i, xref: (0, 0)),
    ),
    out_shape=jax.ShapeDtypeStruct((8, 128), jnp.float32),
    compiler_params=pltpu.CompilerParams(
        dimension_semantics=("arbitrary",),
    ),
)


CHB = 1
NCHB = BPW // CHB
NBUFB = 8
PDB = 4

_SCRATCH_B = [
    pltpu.VMEM((NCHB, CHB), jnp.int32),
] + [pltpu.VMEM((CHB, D), jnp.float32)] * NBUFB + [
    pltpu.VMEM((L,), jnp.float32),
    pltpu.VMEM((4, 128), jnp.float32),
    pltpu.VMEM((8, 128), jnp.float32),
] + [pltpu.SemaphoreType.DMA] * (2 * NBUFB)


@functools.partial(
    pl.kernel,
    out_type=jax.ShapeDtypeStruct((BATCH, D), jnp.float32),
    mesh=_MESH,
    scratch_types=_SCRATCH_B,
)
def _scale_kernel(probs_hbm, x3_hbm, u_hbm, part_hbm, ptc_hbm, out_hbm, *scr):
    idx_v = scr[0]
    bufs = scr[1:1 + NBUFB]
    vec_v = scr[1 + NBUFB]
    part_v = scr[2 + NBUFB]
    ptc_v = scr[3 + NBUFB]
    gsem = scr[4 + NBUFB:4 + 2 * NBUFB]
    wsem = scr[4 + 2 * NBUFB:4 + 3 * NBUFB]
    wid = _wid()
    base = wid * BPW
    pltpu.sync_copy(x3_hbm.at[wid], idx_v)
    pltpu.sync_copy(u_hbm, vec_v)
    pltpu.sync_copy(part_hbm, part_v)
    pltpu.sync_copy(ptc_hbm, ptc_v)
    s_u = _all_sum(vec_v[...])

    tot = jnp.zeros((L,), jnp.float32)
    for r in range(4):
        for jj in range(8):
            tot = tot + part_v[r, pl.ds(jj * L, L)]
    for r in range(8):
        for jj in range(8):
            tot = tot + ptc_v[r, pl.ds(jj * L, L)]
    r_s = 1.0 / _all_sum(tot)

    a2 = (1.0 - s_u) * r_s
    b2c = (s_u * (1.0 / KTAB)) * r_s
    lo = 1e-6 * r_s
    hi = r_s

    def _write(c, buf, sem):
        return pltpu.make_async_copy(
            buf, out_hbm.at[pl.ds(base + c * CHB, CHB)], sem)

    for c in range(PDB):
        _gather(probs_hbm, idx_v, c, bufs[c], gsem[c]).start()

    def step(k, carry):
        for j in range(NBUFB):
            c = NBUFB * k + j
            _gather(probs_hbm, idx_v, c, bufs[j], gsem[j]).wait()

            def body(jj, carry, j=j):
                for q in range(UNROLL):
                    col = (jj * UNROLL + q) * L
                    v = bufs[j][0, pl.ds(col, L)]
                    bufs[j][0, pl.ds(col, L)] = jnp.minimum(
                        jnp.maximum(a2 * v + b2c, lo), hi)
                return carry

            lax.fori_loop(0, D // (L * UNROLL), body, 0)

            _write(c, bufs[j], wsem[j]).start()

            nc = c + PDB
            nj = (j + PDB) % NBUFB

            @pl.when(nc < NCHB)
            def _():
                @pl.when(nc >= NBUFB)
                def _():
                    _write(nc - NBUFB, bufs[nj], wsem[nj]).wait()
                _gather(probs_hbm, idx_v, nc, bufs[nj], gsem[nj]).start()
        return carry

    lax.fori_loop(0, NCHB // NBUFB, step, 0)

    for c in range(NCHB - NBUFB, NCHB):
        _write(c, bufs[c % NBUFB], wsem[c % NBUFB]).wait()


def kernel(probs, x, u, t_now, t_next):
    x32 = x.astype(jnp.int32)
    x3a = x32[:BATCH - TCROWS].reshape(NW, NCHUNK, CH)
    x3b = x32.reshape(NW, NCHB, CHB)
    part = _sum_kernel(probs, x3a, u)
    part_tc = _tc_sum(x32[BATCH - TCROWS:], probs.reshape(KTAB, 1, D),
                      u.reshape(1, L))
    return _scale_kernel(probs, x3b, u, part, part_tc)


# speedup vs baseline: 5.0524x; 5.0524x over previous
import functools

import jax
import jax.numpy as jnp
from jax import lax
from jax.experimental import pallas as pl
from jax.experimental.pallas import tpu as pltpu
from jax.experimental.pallas import tpu_sc as plsc

KTAB = 8192
BATCH = 4096
D = 8192
NC, NS, L = 2, 16, 16
NW = NC * NS
BPW = BATCH // NW
TCROWS = 2048
BPWA = (BATCH - TCROWS) // NW
CH = 2
NCHUNK = BPWA // CH
NBUF = 4
UNROLL = 8

_MESH = plsc.VectorSubcoreMesh(core_axis_name="c", subcore_axis_name="s")


def _wid():
    return lax.axis_index("s") * NC + lax.axis_index("c")


def _all_sum(v):
    for sh in (1, 2, 4, 8):
        idx = (lax.iota(jnp.int32, L) + sh) & (L - 1)
        v = v + v.at[idx].get(mode="promise_in_bounds")
    return v


def _gather(probs_hbm, idx_v, c, buf, sem):
    return pltpu.make_async_copy(probs_hbm.at[idx_v.at[c]], buf, sem)


_SCRATCH = [
    pltpu.VMEM((NCHUNK, CH), jnp.int32),
    pltpu.VMEM((CH, D), jnp.float32),
    pltpu.VMEM((CH, D), jnp.float32),
    pltpu.VMEM((CH, D), jnp.float32),
    pltpu.VMEM((CH, D), jnp.float32),
    pltpu.VMEM((L,), jnp.float32),
    pltpu.SemaphoreType.DMA,
    pltpu.SemaphoreType.DMA,
    pltpu.SemaphoreType.DMA,
    pltpu.SemaphoreType.DMA,
]


@functools.partial(
    pl.kernel,
    out_type=jax.ShapeDtypeStruct((4, 128), jnp.float32),
    mesh=_MESH,
    scratch_types=_SCRATCH,
)
def _sum_kernel(probs_hbm, x3_hbm, u_hbm, part_hbm,
                idx_v, b0, b1, b2, b3, vec_v, g0, g1, g2, g3):
    bufs = (b0, b1, b2, b3)
    gsem = (g0, g1, g2, g3)
    wid = _wid()
    pltpu.sync_copy(x3_hbm.at[wid], idx_v)
    pltpu.sync_copy(u_hbm, vec_v)
    s_u = _all_sum(vec_v[...])
    a = 1.0 - s_u
    b = s_u * (1.0 / KTAB)

    for c in range(NBUF - 1):
        _gather(probs_hbm, idx_v, c, bufs[c], gsem[c]).start()

    def step(k, accs):
        for j in range(NBUF):
            c = NBUF * k + j
            _gather(probs_hbm, idx_v, c, bufs[j], gsem[j]).wait()

            def make_row(r, j=j):
                def body(jj, accs):
                    out = []
                    for q in range(UNROLL):
                        v = bufs[j][r, pl.ds((jj * UNROLL + q) * L, L)]
                        z = jnp.minimum(jnp.maximum(a * v + b, 1e-6), 1.0)
                        out.append(accs[q] + z)
                    return tuple(out)
                return body

            for r in range(CH):
                accs = lax.fori_loop(0, D // (L * UNROLL), make_row(r), accs)

            nc = c + NBUF - 1
            nj = (j + NBUF - 1) % NBUF

            @pl.when(nc < NCHUNK)
            def _():
                _gather(probs_hbm, idx_v, nc, bufs[nj], gsem[nj]).start()
        return accs

    zero = jnp.zeros((L,), jnp.float32)
    accs = lax.fori_loop(0, NCHUNK // NBUF, step, (zero,) * UNROLL)
    acc = accs[0]
    for q in range(1, UNROLL):
        acc = acc + accs[q]
    vec_v[...] = acc
    pltpu.sync_copy(vec_v, part_hbm.at[wid // 8, pl.ds((wid % 8) * L, L)])


TCRPS = 8
TCSTEPS = TCROWS // TCRPS


def _tc_sum_body(x_ref, probs_any, u_ref, acc_ref, buf, sem0, sem1):
    g = pl.program_id(0)
    sems = (sem0, sem1)

    def issue(step, slot):
        for r in range(TCRPS):
            row = x_ref[step * TCRPS + r]
            pltpu.make_async_copy(
                probs_any.at[pl.ds(row, 1)],
                buf.at[slot, pl.ds(r, 1)],
                sems[slot],
            ).start()

    def drain(step, slot):
        for r in range(TCRPS):
            row = x_ref[step * TCRPS + r]
            pltpu.make_async_copy(
                probs_any.at[pl.ds(row, 1)],
                buf.at[slot, pl.ds(r, 1)],
                sems[slot],
            ).wait()

    @pl.when(g == 0)
    def _():
        acc_ref[...] = jnp.zeros((8, 128), jnp.float32)
        issue(0, 0)

    for par in (0, 1):
        @pl.when(jnp.logical_and(g + 1 < TCSTEPS, (g + 1) % 2 == par))
        def _(par=par):
            issue(g + 1, par)

    s_u = jnp.sum(u_ref[...])
    a = 1.0 - s_u
    b = s_u * (1.0 / KTAB)

    for par in (0, 1):
        @pl.when(g % 2 == par)
        def _(par=par):
            drain(g, par)
            v = buf[par]
            z = jnp.minimum(jnp.maximum(a * v + b, 1e-6), 1.0)
            acc_ref[...] += jnp.sum(z.reshape(8, 64, 128), axis=1)


_tc_sum = pl.pallas_call(
    _tc_sum_body,
    grid_spec=pltpu.PrefetchScalarGridSpec(
        num_scalar_prefetch=1,
        grid=(TCSTEPS,),
        in_specs=[
            pl.BlockSpec(memory_space=pl.ANY),
            pl.BlockSpec((1, L), lambda i, xref: (0, 0)),
        ],
        out_specs=pl.BlockSpec((8, 128), lambda i, xref: (0, 0)),
        scratch_shapes=[
            pltpu.VMEM((2, TCRPS, D), jnp.float32),
            pltpu.SemaphoreType.DMA,
            pltpu.SemaphoreType.DMA,
        ],
    ),
    out_shape=jax.ShapeDtypeStruct((8, 128), jnp.float32),
    compiler_params=pltpu.CompilerParams(
        dimension_semantics=("arbitrary",),
    ),
)


CHB = 1
NCHB = BPW // CHB
NBUFB = 8
PDB = 4

_SCRATCH_B = [
    pltpu.VMEM((NCHB, CHB), jnp.int32),
] + [pltpu.VMEM((CHB, D), jnp.float32)] * NBUFB + [
    pltpu.VMEM((L,), jnp.float32),
    pltpu.VMEM((4, 128), jnp.float32),
    pltpu.VMEM((8, 128), jnp.float32),
] + [pltpu.SemaphoreType.DMA] * (2 * NBUFB)


@functools.partial(
    pl.kernel,
    out_type=jax.ShapeDtypeStruct((BATCH, D), jnp.float32),
    mesh=_MESH,
    scratch_types=_SCRATCH_B,
)
def _scale_kernel(probs_hbm, x3_hbm, u_hbm, part_hbm, ptc_hbm, out_hbm, *scr):
    idx_v = scr[0]
    bufs = scr[1:1 + NBUFB]
    vec_v = scr[1 + NBUFB]
    part_v = scr[2 + NBUFB]
    ptc_v = scr[3 + NBUFB]
    gsem = scr[4 + NBUFB:4 + 2 * NBUFB]
    wsem = scr[4 + 2 * NBUFB:4 + 3 * NBUFB]
    wid = _wid()
    base = wid * BPW
    pltpu.sync_copy(x3_hbm.at[wid], idx_v)
    pltpu.sync_copy(u_hbm, vec_v)
    pltpu.sync_copy(part_hbm, part_v)
    pltpu.sync_copy(ptc_hbm, ptc_v)
    s_u = _all_sum(vec_v[...])

    tot = jnp.zeros((L,), jnp.float32)
    for r in range(4):
        for jj in range(8):
            tot = tot + part_v[r, pl.ds(jj * L, L)]
    for r in range(8):
        for jj in range(8):
            tot = tot + ptc_v[r, pl.ds(jj * L, L)]
    r_s = 1.0 / _all_sum(tot)

    a2 = (1.0 - s_u) * r_s
    b2c = (s_u * (1.0 / KTAB)) * r_s
    lo = 1e-6 * r_s
    hi = r_s

    def _write(c, buf, sem):
        return pltpu.make_async_copy(
            buf, out_hbm.at[pl.ds(base + c * CHB, CHB)], sem)

    for c in range(PDB):
        _gather(probs_hbm, idx_v, c, bufs[c], gsem[c]).start()

    def step(k, carry):
        for j in range(NBUFB):
            c = NBUFB * k + j
            _gather(probs_hbm, idx_v, c, bufs[j], gsem[j]).wait()

            def body(jj, carry, j=j):
                for q in range(UNROLL):
                    col = (jj * UNROLL + q) * L
                    v = bufs[j][0, pl.ds(col, L)]
                    bufs[j][0, pl.ds(col, L)] = jnp.minimum(
                        jnp.maximum(a2 * v + b2c, lo), hi)
                return carry

            lax.fori_loop(0, D // (L * UNROLL), body, 0)

            _write(c, bufs[j], wsem[j]).start()

            nc = c + PDB
            nj = (j + PDB) % NBUFB

            @pl.when(nc < NCHB)
            def _():
                @pl.when(nc >= NBUFB)
                def _():
                    _write(nc - NBUFB, bufs[nj], wsem[nj]).wait()
                _gather(probs_hbm, idx_v, nc, bufs[nj], gsem[nj]).start()
        return carry

    lax.fori_loop(0, NCHB // NBUFB, step, 0)

    for c in range(NCHB - NBUFB, NCHB):
        _write(c, bufs[c % NBUFB], wsem[c % NBUFB]).wait()


def kernel(probs, x, u, t_now, t_next):
    x32 = x.astype(jnp.int32)
    x3a = x32[:BATCH - TCROWS].reshape(NW, NCHUNK, CH)
    x3b = x32.reshape(NW, NCHB, CHB)
    part = _sum_kernel(probs, x3a, u)
    part_tc = _tc_sum(x32[BATCH - TCROWS:], probs, u.reshape(1, L))
    return _scale_kernel(probs, x3b, u, part, part_tc)
